# baseline (speedup 1.0000x reference)
import functools

import jax
import jax.numpy as jnp
from jax import lax
from jax.experimental import pallas as pl
from jax.experimental.pallas import tpu as pltpu
from jax.experimental.pallas import tpu_sc as plsc

_PREC = lax.Precision.DEFAULT


_NB_PROJ = 1024


def _proj_body(e_ref, pw_ref, pb_ref, qc_ref, cn_ref):
    qc = lax.dot_general(
        e_ref[...], pw_ref[...], (((1,), (1,)), ((), ())),
        preferred_element_type=jnp.float32, precision=_PREC) + pb_ref[...]
    qc_ref[...] = qc
    cn_ref[...] = jnp.sum(qc * qc, axis=1, keepdims=True)


def _project(embed_w, proj_w, proj_b):
    n_embed, dim = embed_w.shape
    nb = _NB_PROJ
    qc, cn_col = pl.pallas_call(
        _proj_body,
        grid=(n_embed // nb,),
        in_specs=[
            pl.BlockSpec((nb, dim), lambda i: (i, 0)),
            pl.BlockSpec((dim, dim), lambda i: (0, 0)),
            pl.BlockSpec((1, dim), lambda i: (0, 0)),
        ],
        out_specs=[
            pl.BlockSpec((nb, dim), lambda i: (i, 0)),
            pl.BlockSpec((nb, 1), lambda i: (i, 0)),
        ],
        out_shape=[
            jax.ShapeDtypeStruct((n_embed, dim), jnp.float32),
            jax.ShapeDtypeStruct((n_embed, 1), jnp.float32),
        ],
    )(embed_w, proj_w, proj_b.reshape(1, dim))
    return qc, cn_col


_BM = 512
_BN = 1024


def _dist_body(x_ref, qc_ref, cn_ref, idx_ref, xs_ref, xn_ref, rmin_ref,
               ridx_ref):
    n = pl.program_id(1)
    n_last = pl.num_programs(1) - 1

    @pl.when(n == 0)
    def _():
        x = x_ref[...]
        xs_ref[...] = x * -2.0
        xn_ref[...] = jnp.sum(x * x, axis=1, keepdims=True)

    m2 = lax.dot_general(
        xs_ref[...], qc_ref[...], (((1,), (1,)), ((), ())),
        preferred_element_type=jnp.float32, precision=_PREC)
    s = (xn_ref[...] + cn_ref[...]) + m2
    rowmin = jnp.min(s, axis=1, keepdims=True)
    iota = lax.broadcasted_iota(jnp.int32, s.shape, 1) + n * s.shape[1]
    cand = jnp.where(s == rowmin, iota, jnp.int32(2**31 - 1))
    bidx = jnp.min(cand, axis=1, keepdims=True)

    @pl.when(n == 0)
    def _():
        rmin_ref[...] = rowmin
        ridx_ref[...] = bidx

    @pl.when(n > 0)
    def _():
        better = rowmin < rmin_ref[...]
        ridx_ref[...] = jnp.where(better, bidx, ridx_ref[...])
        rmin_ref[...] = jnp.where(better, rowmin, rmin_ref[...])

    @pl.when(n == n_last)
    def _():
        idx_ref[...] = ridx_ref[...]


def _distargmin(flat, qc, cn_col):
    m, dim = flat.shape
    n_embed = qc.shape[0]
    cn_row = cn_col.reshape(1, n_embed)
    idx2 = pl.pallas_call(
        _dist_body,
        grid=(m // _BM, n_embed // _BN),
        in_specs=[
            pl.BlockSpec((_BM, dim), lambda i, j: (i, 0)),
            pl.BlockSpec((_BN, dim), lambda i, j: (j, 0)),
            pl.BlockSpec((1, _BN), lambda i, j: (0, j)),
        ],
        out_specs=pl.BlockSpec((_BM, 1), lambda i, j: (i, 0)),
        out_shape=jax.ShapeDtypeStruct((m, 1), jnp.int32),
        scratch_shapes=[
            pltpu.VMEM((_BM, dim), jnp.float32),
            pltpu.VMEM((_BM, 1), jnp.float32),
            pltpu.VMEM((_BM, 1), jnp.float32),
            pltpu.VMEM((_BM, 1), jnp.int32),
        ],
    )(flat, qc, cn_row)
    return idx2.reshape(m)


def _gather(qc, idx):
    m = idx.shape[0]
    n_embed, dim = qc.shape
    info = plsc.get_sparse_core_info()
    nc, ns = info.num_cores, info.num_subcores
    nw = nc * ns
    b_per_w = m // nw
    chunk = b_per_w
    while chunk * dim * 4 > 256 * 1024 or chunk % 8:
        for c in range(chunk - 1, 0, -1):
            if b_per_w % c == 0:
                chunk = c
                break
        else:
            chunk = 8
            break
    n_chunks = b_per_w // chunk
    mesh = plsc.VectorSubcoreMesh(core_axis_name="c", subcore_axis_name="s")

    @functools.partial(
        pl.kernel, mesh=mesh,
        out_type=jax.ShapeDtypeStruct((m, dim), jnp.float32),
        scratch_types=[
            pltpu.VMEM((chunk,), jnp.int32),
            pltpu.VMEM((chunk, dim), jnp.float32),
            pltpu.SemaphoreType.DMA,
        ],
    )
    def _k(table_hbm, idx_hbm, out_hbm, idx_v, rows_v, sem):
        wid = lax.axis_index("s") * nc + lax.axis_index("c")
        base = wid * b_per_w
        for c in range(n_chunks):
            off = base + c * chunk
            pltpu.sync_copy(idx_hbm.at[pl.ds(off, chunk)], idx_v)
            pltpu.async_copy(table_hbm.at[idx_v], rows_v, sem).wait()
            pltpu.sync_copy(rows_v, out_hbm.at[pl.ds(off, chunk)])

    return _k(qc, idx)


_BD = 512


def _st_body(z_ref, zq_ref, st_ref, diff_ref, acc_ref):
    i = pl.program_id(0)
    i_last = pl.num_programs(0) - 1
    n_total = pl.num_programs(0) * z_ref.shape[0] * z_ref.shape[1]
    z = z_ref[...]
    d = zq_ref[...] - z
    st_ref[...] = z + d
    ps = jnp.sum(d * d)

    @pl.when(i == 0)
    def _():
        acc_ref[0] = ps

    @pl.when(i > 0)
    def _():
        acc_ref[0] = acc_ref[0] + ps

    @pl.when(i == i_last)
    def _():
        m1 = acc_ref[0] / jnp.float32(n_total)
        diff_ref[...] = jnp.reshape(m1 + 0.25 * m1, (1, 1))


def _finish(flat, zq):
    m, dim = flat.shape
    st, diff = pl.pallas_call(
        _st_body,
        grid=(m // _BD,),
        in_specs=[
            pl.BlockSpec((_BD, dim), lambda i: (i, 0)),
            pl.BlockSpec((_BD, dim), lambda i: (i, 0)),
        ],
        out_specs=[
            pl.BlockSpec((_BD, dim), lambda i: (i, 0)),
            pl.BlockSpec((1, 1), lambda i: (0, 0)),
        ],
        out_shape=[
            jax.ShapeDtypeStruct((m, dim), jnp.float32),
            jax.ShapeDtypeStruct((1, 1), jnp.float32),
        ],
        scratch_shapes=[pltpu.SMEM((1,), jnp.float32)],
    )(flat, zq)
    return st, diff


def kernel(z, embed_w, proj_w, proj_b):
    dim = embed_w.shape[1]
    flat = z.reshape(-1, dim)
    qc, cn_col = _project(embed_w, proj_w, proj_b)
    idx = _distargmin(flat, qc, cn_col)
    zq = _gather(qc, idx)
    st, diff = _finish(flat, zq)
    return st.reshape(z.shape), diff.reshape(()), idx


# speedup vs baseline: 1.1503x; 1.1503x over previous
import functools

import jax
import jax.numpy as jnp
from jax import lax
from jax.experimental import pallas as pl
from jax.experimental.pallas import tpu as pltpu
from jax.experimental.pallas import tpu_sc as plsc

_PREC = lax.Precision.DEFAULT

_BM = 512
_BN = 1024
_BC = 256


def _dist_body(e_hbm, pw_ref, pb_ref, x_ref, qc_ref, idx_ref,
               cn_ref, ebuf, xs_ref, xn_ref, rmin_ref, ridx_ref, sem):
    mi = pl.program_id(0)
    n = pl.program_id(1)
    n_last = pl.num_programs(1) - 1
    bn = ebuf.shape[0]

    @pl.when(mi == 0)
    def _():
        cp = pltpu.make_async_copy(e_hbm.at[pl.ds(n * bn, bn), :], ebuf, sem)
        cp.start()
        cp.wait()
        qc = lax.dot_general(
            ebuf[...], pw_ref[...], (((1,), (1,)), ((), ())),
            preferred_element_type=jnp.float32, precision=_PREC) + pb_ref[...]
        qc_ref[pl.ds(n * bn, bn), :] = qc
        cn_ref[:, pl.ds(n * bn, bn)] = jnp.sum(
            qc * qc, axis=1, keepdims=True).reshape(1, bn)

    @pl.when(n == 0)
    def _():
        x = x_ref[...]
        xs_ref[...] = x * -2.0
        xn_ref[...] = jnp.sum(x * x, axis=1, keepdims=True)

    xs = xs_ref[...]
    xn = xn_ref[...]

    rm_run = None
    bi_run = None
    for c in range(bn // _BC):
        qc_c = qc_ref[pl.ds(n * bn + c * _BC, _BC), :]
        m2 = lax.dot_general(
            xs, qc_c, (((1,), (1,)), ((), ())),
            preferred_element_type=jnp.float32, precision=_PREC)
        s = (xn + cn_ref[:, pl.ds(n * bn + c * _BC, _BC)]) + m2
        rm = jnp.min(s, axis=1, keepdims=True)
        iot = lax.broadcasted_iota(jnp.int32, s.shape, 1).astype(jnp.float32)
        cand = jnp.where(s == rm, iot, jnp.float32(jnp.inf))
        bi = jnp.min(cand, axis=1, keepdims=True) + (
            (n * bn + c * _BC).astype(jnp.float32))
        if rm_run is None:
            rm_run, bi_run = rm, bi
        else:
            better = rm < rm_run
            bi_run = jnp.where(better, bi, bi_run)
            rm_run = jnp.where(better, rm, rm_run)

    @pl.when(n == 0)
    def _():
        rmin_ref[...] = rm_run
        ridx_ref[...] = bi_run

    @pl.when(n > 0)
    def _():
        better = rm_run < rmin_ref[...]
        ridx_ref[...] = jnp.where(better, bi_run, ridx_ref[...])
        rmin_ref[...] = jnp.where(better, rm_run, rmin_ref[...])

    @pl.when(n == n_last)
    def _():
        idx_ref[...] = ridx_ref[...].astype(jnp.int32)


def _distargmin(flat, embed_w, proj_w, proj_b):
    m, dim = flat.shape
    n_embed = embed_w.shape[0]
    qc, idx2 = pl.pallas_call(
        _dist_body,
        grid=(m // _BM, n_embed // _BN),
        in_specs=[
            pl.BlockSpec(memory_space=pltpu.MemorySpace.HBM),
            pl.BlockSpec((dim, dim), lambda i, n: (0, 0)),
            pl.BlockSpec((1, dim), lambda i, n: (0, 0)),
            pl.BlockSpec((_BM, dim), lambda i, n: (i, 0)),
        ],
        out_specs=[
            pl.BlockSpec((n_embed, dim), lambda i, n: (0, 0)),
            pl.BlockSpec((_BM, 1), lambda i, n: (i, 0)),
        ],
        out_shape=[
            jax.ShapeDtypeStruct((n_embed, dim), jnp.float32),
            jax.ShapeDtypeStruct((m, 1), jnp.int32),
        ],
        scratch_shapes=[
            pltpu.VMEM((1, n_embed), jnp.float32),
            pltpu.VMEM((_BN, dim), jnp.float32),
            pltpu.VMEM((_BM, dim), jnp.float32),
            pltpu.VMEM((_BM, 1), jnp.float32),
            pltpu.VMEM((_BM, 1), jnp.float32),
            pltpu.VMEM((_BM, 1), jnp.float32),
            pltpu.SemaphoreType.DMA,
        ],
    )(embed_w, proj_w, proj_b.reshape(1, dim), flat)
    return qc, idx2.reshape(m)


def _gather(qc, idx):
    m = idx.shape[0]
    n_embed, dim = qc.shape
    info = plsc.get_sparse_core_info()
    nc, ns = info.num_cores, info.num_subcores
    nw = nc * ns
    b_per_w = m // nw
    chunk = b_per_w
    while chunk * dim * 4 > 256 * 1024 or chunk % 8:
        for c in range(chunk - 1, 0, -1):
            if b_per_w % c == 0:
                chunk = c
                break
        else:
            chunk = 8
            break
    n_chunks = b_per_w // chunk
    mesh = plsc.VectorSubcoreMesh(core_axis_name="c", subcore_axis_name="s")

    @functools.partial(
        pl.kernel, mesh=mesh,
        out_type=jax.ShapeDtypeStruct((m, dim), jnp.float32),
        scratch_types=[
            pltpu.VMEM((chunk,), jnp.int32),
            pltpu.VMEM((chunk, dim), jnp.float32),
            pltpu.SemaphoreType.DMA,
        ],
    )
    def _k(table_hbm, idx_hbm, out_hbm, idx_v, rows_v, sem):
        wid = lax.axis_index("s") * nc + lax.axis_index("c")
        base = wid * b_per_w
        for c in range(n_chunks):
            off = base + c * chunk
            pltpu.sync_copy(idx_hbm.at[pl.ds(off, chunk)], idx_v)
            pltpu.async_copy(table_hbm.at[idx_v], rows_v, sem).wait()
            pltpu.sync_copy(rows_v, out_hbm.at[pl.ds(off, chunk)])

    return _k(qc, idx)


_BD = 512


def _st_body(z_ref, zq_ref, st_ref, diff_ref, acc_ref):
    i = pl.program_id(0)
    i_last = pl.num_programs(0) - 1
    n_total = pl.num_programs(0) * z_ref.shape[0] * z_ref.shape[1]
    z = z_ref[...]
    d = zq_ref[...] - z
    st_ref[...] = z + d
    ps = jnp.sum(d * d)

    @pl.when(i == 0)
    def _():
        acc_ref[0] = ps

    @pl.when(i > 0)
    def _():
        acc_ref[0] = acc_ref[0] + ps

    @pl.when(i == i_last)
    def _():
        m1 = acc_ref[0] / jnp.float32(n_total)
        diff_ref[...] = jnp.reshape(m1 + 0.25 * m1, (1, 1))


def _finish(flat, zq):
    m, dim = flat.shape
    st, diff = pl.pallas_call(
        _st_body,
        grid=(m // _BD,),
        in_specs=[
            pl.BlockSpec((_BD, dim), lambda i: (i, 0)),
            pl.BlockSpec((_BD, dim), lambda i: (i, 0)),
        ],
        out_specs=[
            pl.BlockSpec((_BD, dim), lambda i: (i, 0)),
            pl.BlockSpec((1, 1), lambda i: (0, 0)),
        ],
        out_shape=[
            jax.ShapeDtypeStruct((m, dim), jnp.float32),
            jax.ShapeDtypeStruct((1, 1), jnp.float32),
        ],
        scratch_shapes=[pltpu.SMEM((1,), jnp.float32)],
    )(flat, zq)
    return st, diff


def kernel(z, embed_w, proj_w, proj_b):
    dim = embed_w.shape[1]
    flat = z.reshape(-1, dim)
    qc, idx = _distargmin(flat, embed_w, proj_w, proj_b)
    zq = _gather(qc, idx)
    st, diff = _finish(flat, zq)
    return st.reshape(z.shape), diff.reshape(()), idx
